# Initial kernel scaffold; baseline (speedup 1.0000x reference)
import jax
import jax.numpy as jnp
from jax.experimental import pallas as pl


def kernel(x, edge_index, edge_weight_params, W1, b1, W2, b2, W3, b3):
    raise NotImplementedError("write your pallas kernel here")


# speedup vs baseline: 1.0833x; 1.0833x over previous
import jax
import jax.numpy as jnp
from jax.experimental import pallas as pl


def _mm_kernel(x_ref, w_ref, b_ref, o_ref):
    o_ref[...] = jnp.dot(x_ref[...], w_ref[...],
                         preferred_element_type=jnp.float32) + b_ref[...]


def _matmul(x, W, b):
    n, d = x.shape
    dout = W.shape[1]
    if dout % 128:
        pad = 128 - dout % 128
        W = jnp.pad(W, ((0, 0), (0, pad)))
        b = jnp.pad(b, ((0, pad),))
    dp = W.shape[1]
    b2 = b.reshape(1, dp)
    blk = 1000
    out = pl.pallas_call(
        _mm_kernel,
        grid=(n // blk,),
        in_specs=[
            pl.BlockSpec((blk, d), lambda i: (i, 0)),
            pl.BlockSpec((d, dp), lambda i: (0, 0)),
            pl.BlockSpec((1, dp), lambda i: (0, 0)),
        ],
        out_specs=pl.BlockSpec((blk, dp), lambda i: (i, 0)),
        out_shape=jax.ShapeDtypeStruct((n, dp), jnp.float32),
    )(x, W, b2)
    return out[:, :dout]


def kernel(x, edge_index, edge_weight_params, W1, b1, W2, b2, W3, b3):
    n = x.shape[0]
    src = edge_index[0]
    dst = edge_index[1]
    w = jax.nn.sigmoid(edge_weight_params)
    loop = jnp.arange(n, dtype=src.dtype)
    src_f = jnp.concatenate([src, loop])
    dst_f = jnp.concatenate([dst, loop])
    ew = jnp.concatenate([w, jnp.ones((n,), dtype=w.dtype)])

    def conv(h_in, W, b):
        deg = jnp.zeros((n,), dtype=jnp.float32).at[dst_f].add(ew)
        dinv = jnp.where(deg > 0, 1.0 / jnp.sqrt(jnp.maximum(deg, 1e-12)), 0.0)
        norm = dinv[src_f] * ew * dinv[dst_f]
        h = _matmul(h_in, W, b * 0.0)
        msg = h[src_f] * norm[:, None]
        out = jnp.zeros((n, W.shape[1]), dtype=jnp.float32).at[dst_f].add(msg)
        return out + b

    h1 = jax.nn.relu(conv(x, W1, b1))
    h2 = jax.nn.relu(conv(h1, W2, b2))
    h3 = conv(h2, W3, b3)
    return jax.nn.log_softmax(h3, axis=-1)[0]


# speedup vs baseline: 10.1016x; 9.3251x over previous
import functools

import jax
import jax.numpy as jnp
from jax import lax
from jax.experimental import pallas as pl
from jax.experimental.pallas import tpu as pltpu
from jax.experimental.pallas import tpu_sc as plsc

N_NODES = 10000
NPAD = 10240
E = 320000
EPT = 20096
NCH = 157
EPAD = EPT * 16
SLC = 640
SRC_SH = 14
M14 = (1 << 14) - 1
FBIT = 1 << 28
CCAP = EPAD + 2048
CZ = CCAP // 16 // 8


def _sc_mesh():
    return plsc.VectorSubcoreMesh(core_axis_name="c", subcore_axis_name="s",
                                  num_cores=1)


def _prep(pk3, ewp3):
    f32 = jnp.float32
    i32 = jnp.int32
    out_type = [
        jax.ShapeDtypeStruct((NPAD,), f32),
        jax.ShapeDtypeStruct((NPAD,), f32),
        jax.ShapeDtypeStruct((EPAD,), i32),
        jax.ShapeDtypeStruct((EPAD,), f32),
        jax.ShapeDtypeStruct((16, 16), i32),
        jax.ShapeDtypeStruct((NPAD,), f32),
        jax.ShapeDtypeStruct((NPAD,), f32),
    ]
    scratch = [
        pltpu.VMEM((NCH, 128), i32),
        pltpu.VMEM((NCH, 128), i32),
        pltpu.VMEM((NCH, 128), f32),
        pltpu.VMEM((1, 128), i32),
        pltpu.VMEM((1, 128), i32),
        pltpu.VMEM((2528,), i32),
        pltpu.VMEM((2528,), f32),
        pltpu.VMEM((SLC,), f32),
        pltpu.VMEM((SLC,), f32),
        pltpu.VMEM((128,), f32),
        pltpu.VMEM((128,), f32),
        pltpu.VMEM((128,), f32),
        pltpu.VMEM((16,), i32),
        pltpu.VMEM_SHARED((NPAD,), f32),
        pltpu.VMEM_SHARED((NPAD,), f32),
        pltpu.VMEM_SHARED((NPAD,), f32),
        pltpu.VMEM_SHARED((CCAP,), i32),
        pltpu.VMEM_SHARED((CCAP,), f32),
        pltpu.SemaphoreType.DMA,
    ]

    @functools.partial(pl.kernel, out_type=out_type, mesh=_sc_mesh(),
                       scratch_types=scratch)
    def k(pk_h, ewp_h, dinv_h, a2_h, e1p_h, e1n_h, cnt_h,
          m1_h, m2_h, srcV, dstV, nrmV, posW, iW2, zbufI, zbufF,
          sliceA, sliceB, gA, gB, vB, cntT, degS, a2S, m1S,
          cPkS, cNrmS, sem):
        tid = lax.axis_index("s")
        nsl = pl.ds(tid * SLC, SLC)
        ones16 = jnp.ones((16,), f32)
        zf16 = jnp.zeros((16,), f32)
        zi16 = jnp.zeros((16,), i32)

        pltpu.sync_copy(pk_h.at[tid], srcV)
        pltpu.sync_copy(ewp_h.at[tid], nrmV)

        def _unpack(j, _):
            for kk in range(8):
                d = pl.ds(kk * 16, 16)
                v = srcV[j, d]
                dstV[j, d] = v & M14
                srcV[j, d] = lax.shift_right_logical(v, SRC_SH)
            return 0
        lax.fori_loop(0, NCH, _unpack, 0)

        def _fill(i, _):
            sliceA[pl.ds(i * 16, 16)] = ones16
            sliceB[pl.ds(i * 16, 16)] = zf16
            return 0
        lax.fori_loop(0, SLC // 16, _fill, 0)
        pltpu.sync_copy(sliceA, degS.at[nsl])
        pltpu.sync_copy(sliceB, a2S.at[nsl])
        pltpu.sync_copy(sliceB, m1S.at[nsl])

        def _zero_z(i, _):
            d = pl.ds(i * 16, 16)
            zbufI[d] = zi16
            zbufF[d] = zf16
            return 0
        lax.fori_loop(0, CZ // 16, _zero_z, 0)
        for m in range(8):
            cz = pl.ds(tid * (CCAP // 16) + m * CZ, CZ)
            pltpu.sync_copy(zbufI, cPkS.at[cz])
            pltpu.sync_copy(zbufF, cNrmS.at[cz])
        plsc.subcore_barrier()

        def _p1(j, _):
            for kk in range(8):
                d = pl.ds(kk * 16, 16)
                p = nrmV[j, d]
                nrmV[j, d] = 1.0 / (1.0 + jnp.exp(-p))
            pltpu.sync_copy(nrmV.at[j], degS.at[dstV.at[j]], add=True)
            return 0
        lax.fori_loop(0, NCH, _p1, 0)
        plsc.subcore_barrier()

        pltpu.sync_copy(degS.at[nsl], sliceA)

        def _p2(i, _):
            d = pl.ds(i * 16, 16)
            x = sliceA[d]
            iv = lax.bitcast_convert_type(x, i32)
            iv = 0x5F3759DF - lax.shift_right_arithmetic(iv, 1)
            y = lax.bitcast_convert_type(iv, f32)
            y = y * (1.5 - 0.5 * x * y * y)
            y = y * (1.5 - 0.5 * x * y * y)
            y = y * (1.5 - 0.5 * x * y * y)
            sliceA[d] = y
            return 0
        lax.fori_loop(0, SLC // 16, _p2, 0)
        pltpu.sync_copy(sliceA, dinv_h.at[nsl])
        plsc.subcore_barrier()

        def _p3(j, _):
            pltpu.async_copy(dinv_h.at[srcV.at[j]], gA, sem).wait()
            pltpu.async_copy(dinv_h.at[dstV.at[j]], gB, sem).wait()
            for kk in range(8):
                d = pl.ds(kk * 16, 16)
                n = gA[d] * nrmV[j, d] * gB[d]
                nrmV[j, d] = n
                vB[d] = jnp.where(dstV[j, d] == 0, n, 0.0)
            pltpu.sync_copy(vB, a2S.at[srcV.at[j]], add=True)
            return 0
        lax.fori_loop(0, NCH, _p3, 0)
        plsc.subcore_barrier()

        pltpu.sync_copy(a2S.at[nsl], sliceA)

        def _p4(i, _):
            d = pl.ds(i * 16, 16)
            vid = lax.iota(i32, 16) + (tid * SLC + i * 16)
            a = sliceA[d]
            sliceB[d] = jnp.where((a > 0.0) | (vid == 0), 1.0, 0.0)
            return 0
        lax.fori_loop(0, SLC // 16, _p4, 0)
        pltpu.sync_copy(sliceA, a2_h.at[nsl])
        pltpu.sync_copy(sliceB, m2_h.at[nsl])
        plsc.subcore_barrier()

        def _p5(j, _):
            pltpu.async_copy(m2_h.at[dstV.at[j]], vB, sem).wait()
            pltpu.sync_copy(vB, m1S.at[srcV.at[j]], add=True)
            return 0
        lax.fori_loop(0, NCH, _p5, 0)
        plsc.subcore_barrier()

        pltpu.sync_copy(m1S.at[nsl], sliceA)
        pltpu.sync_copy(m2_h.at[nsl], sliceB)

        def _p6(i, _):
            d = pl.ds(i * 16, 16)
            sliceA[d] = sliceA[d] + sliceB[d]
            return 0
        lax.fori_loop(0, SLC // 16, _p6, 0)
        pltpu.sync_copy(sliceA, m1_h.at[nsl])
        plsc.subcore_barrier()

        lane = lax.iota(i32, 16)
        gdn = lax.GatherDimensionNumbers(offset_dims=(),
                                         collapsed_slice_dims=(0,),
                                         start_index_map=(0,))

        def _p7(j, cnt):
            pltpu.async_copy(m1_h.at[dstV.at[j]], gA, sem).wait()
            pltpu.async_copy(m2_h.at[dstV.at[j]], gB, sem).wait()
            for kk in range(8):
                d = pl.ds(kk * 16, 16)
                act = gA[d] > 0.0
                cs = jnp.where(act, 1, 0)
                for sh in (1, 2, 4, 8):
                    g = lax.gather(cs, jnp.maximum(lane - sh, 0).reshape(16, 1),
                                   gdn, (1,),
                                   mode=lax.GatherScatterMode.PROMISE_IN_BOUNDS)
                    cs = cs + jnp.where(lane >= sh, g, 0)
                dump = EPAD + (j % 16) * 128 + kk * 16 + lane
                pos = jnp.where(act, tid * EPT + cnt + cs - 1, dump)
                posW[0, d] = pos
                pk = srcV[j, d] * (M14 + 1) + dstV[j, d]
                iW2[0, d] = pk + jnp.where(gB[d] > 0.0, FBIT, 0)
                cnt = cnt + cs[15]
            pltpu.sync_copy(iW2.at[0], cPkS.at[posW.at[0]], add=True)
            pltpu.sync_copy(nrmV.at[j], cNrmS.at[posW.at[0]], add=True)
            return cnt
        cnt = lax.fori_loop(0, NCH, _p7, jnp.int32(0))
        plsc.subcore_barrier()

        esl = pl.ds(tid * EPT, EPT)
        pltpu.sync_copy(cPkS.at[esl], e1p_h.at[esl])
        pltpu.sync_copy(cNrmS.at[esl], e1n_h.at[esl])
        cntT[pl.ds(0, 16)] = jnp.full((16,), cnt, i32)
        pltpu.sync_copy(cntT, cnt_h.at[tid])

    return k(pk3, ewp3)


def _edge_layer(h, e1p, e1n, cnt, layer2):
    f32 = jnp.float32
    i32 = jnp.int32
    scratch = [
        pltpu.VMEM((128,), i32),
        pltpu.VMEM((1, 128), i32),
        pltpu.VMEM((1, 128), i32),
        pltpu.VMEM((144,), f32),
        pltpu.VMEM((128, 128), f32),
        pltpu.VMEM((16,), i32),
        pltpu.VMEM_SHARED((NPAD, 128), f32),
        pltpu.SemaphoreType.DMA,
    ]

    @functools.partial(pl.kernel,
                       out_type=[jax.ShapeDtypeStruct((NPAD, 128), f32)],
                       mesh=_sc_mesh(), scratch_types=scratch)
    def k(h_h, e1p_h, e1n_h, cnt_h, acc_h,
          srcC, idxW, pkC, nrmC, rowsB, cntT, accS, sem):
        tid = lax.axis_index("s")
        zf16 = jnp.zeros((16,), f32)

        def _zr(r, _):
            for kk in range(8):
                rowsB[r, pl.ds(kk * 16, 16)] = zf16
            return 0
        lax.fori_loop(0, 128, _zr, 0)
        for m in range(SLC // 128):
            pltpu.sync_copy(rowsB, accS.at[pl.ds(tid * SLC + m * 128, 128)])
        plsc.subcore_barrier()

        pltpu.sync_copy(cnt_h.at[tid], cntT)
        cnt = cntT[pl.ds(0, 16)][0]
        nch = (cnt + 127) // 128

        def _chunk(j, _):
            base = tid * EPT + j * 128
            pltpu.sync_copy(e1p_h.at[pl.ds(base, 128)], pkC.at[0])
            pltpu.sync_copy(e1n_h.at[pl.ds(base, 128)], nrmC.at[pl.ds(0, 128)])
            for kk in range(8):
                d = pl.ds(kk * 16, 16)
                v = pkC[0, d]
                srcC[d] = lax.shift_right_logical(v, SRC_SH) & M14
                idxW[0, d] = v & M14
                if layer2:
                    nrmC[d] = jnp.where(v >= FBIT, nrmC[d], 0.0)
            pltpu.async_copy(h_h.at[srcC], rowsB, sem).wait()

            def _scale(r, _):
                s = nrmC[pl.ds(r, 16)][0]
                for kk in range(8):
                    d = pl.ds(kk * 16, 16)
                    rowsB[r, d] = rowsB[r, d] * s
                return 0
            lax.fori_loop(0, 128, _scale, 0)
            pltpu.sync_copy(rowsB, accS.at[idxW.at[0]], add=True)
            return 0
        lax.fori_loop(0, nch, _chunk, 0)
        plsc.subcore_barrier()

        for m in range(SLC // 128):
            rs = pl.ds(tid * SLC + m * 128, 128)
            pltpu.sync_copy(accS.at[rs], rowsB)
            pltpu.sync_copy(rowsB, acc_h.at[rs])

    return k(h, e1p, e1n, cnt)[0]


def _mm_kernel(x_ref, w_ref, o_ref):
    o_ref[...] = jnp.dot(x_ref[...], w_ref[...],
                         preferred_element_type=jnp.float32)


def _tc_matmul(x, W):
    n, d = x.shape
    dout = W.shape[1]
    blk = 1280
    return pl.pallas_call(
        _mm_kernel,
        grid=(n // blk,),
        in_specs=[pl.BlockSpec((blk, d), lambda i: (i, 0)),
                  pl.BlockSpec((d, dout), lambda i: (0, 0))],
        out_specs=pl.BlockSpec((blk, dout), lambda i: (i, 0)),
        out_shape=jax.ShapeDtypeStruct((n, dout), jnp.float32),
    )(x, W)


def _mid_kernel(acc_ref, hp_ref, di_ref, b_ref, w_ref, o_ref):
    d2 = di_ref[...] * di_ref[...]
    z = jnp.maximum(acc_ref[...] + d2 * hp_ref[...] + b_ref[...], 0.0)
    o_ref[...] = jnp.dot(z, w_ref[...], preferred_element_type=jnp.float32)


def _tc_mid(acc, hp, dinv, b, W):
    n, d = acc.shape
    blk = 1280
    return pl.pallas_call(
        _mid_kernel,
        grid=(n // blk,),
        in_specs=[pl.BlockSpec((blk, d), lambda i: (i, 0)),
                  pl.BlockSpec((blk, d), lambda i: (i, 0)),
                  pl.BlockSpec((blk, 1), lambda i: (i, 0)),
                  pl.BlockSpec((1, d), lambda i: (0, 0)),
                  pl.BlockSpec((d, d), lambda i: (0, 0))],
        out_specs=pl.BlockSpec((blk, d), lambda i: (i, 0)),
        out_shape=jax.ShapeDtypeStruct((n, d), jnp.float32),
    )(acc, hp, dinv, b, W)


def _fin_kernel(acc_ref, hp_ref, di_ref, b_ref, a2_ref, w3_ref, b3_ref,
                o_ref, scr_ref):
    i = pl.program_id(0)
    d2 = di_ref[...] * di_ref[...]
    z = jnp.maximum(acc_ref[...] + d2 * hp_ref[...] + b_ref[...], 0.0)
    pv = jnp.sum(a2_ref[...] * z, axis=0, keepdims=True)

    @pl.when(i == 0)
    def _():
        scr_ref[...] = pv

    @pl.when(i > 0)
    def _():
        scr_ref[...] = scr_ref[...] + pv

    @pl.when(i == pl.num_programs(0) - 1)
    def _():
        logits = jnp.dot(scr_ref[...], w3_ref[...],
                         preferred_element_type=jnp.float32) + b3_ref[...]
        mx = jnp.max(logits)
        ls = logits - mx
        o_ref[...] = ls - jnp.log(jnp.sum(jnp.exp(ls)))


def _tc_fin(acc, hp, dinv, b, a2c, W3, b3):
    n, d = acc.shape
    dout = W3.shape[1]
    blk = 1280
    return pl.pallas_call(
        _fin_kernel,
        grid=(n // blk,),
        in_specs=[pl.BlockSpec((blk, d), lambda i: (i, 0)),
                  pl.BlockSpec((blk, d), lambda i: (i, 0)),
                  pl.BlockSpec((blk, 1), lambda i: (i, 0)),
                  pl.BlockSpec((1, d), lambda i: (0, 0)),
                  pl.BlockSpec((blk, 1), lambda i: (i, 0)),
                  pl.BlockSpec((d, dout), lambda i: (0, 0)),
                  pl.BlockSpec((1, dout), lambda i: (0, 0))],
        out_specs=pl.BlockSpec((1, dout), lambda i: (0, 0)),
        out_shape=jax.ShapeDtypeStruct((1, dout), jnp.float32),
        scratch_shapes=[pltpu.VMEM((1, d), jnp.float32)],
    )(acc, hp, dinv, b, a2c, W3, b3)


def kernel(x, edge_index, edge_weight_params, W1, b1, W2, b2, W3, b3):
    i32 = jnp.int32
    f32 = jnp.float32
    npe = EPAD - E
    pk = edge_index[0].astype(i32) * (M14 + 1) + edge_index[1].astype(i32)
    pk = jnp.concatenate([pk, jnp.zeros((npe,), i32)])
    ewp = jnp.concatenate([edge_weight_params.astype(f32),
                           jnp.full((npe,), -1e9, f32)])
    pk3 = pk.reshape(16, NCH, 128)
    ewp3 = ewp.reshape(16, NCH, 128)

    dinv, a2, e1p, e1n, cnt, _m1, _m2 = _prep(pk3, ewp3)

    xpad = jnp.pad(x, ((0, NPAD - N_NODES), (0, 0)))
    h1p = _tc_matmul(xpad, W1)

    acc1 = _edge_layer(h1p, e1p, e1n, cnt, layer2=False)
    dcol = dinv.reshape(NPAD, 1)
    g2 = _tc_mid(acc1, h1p, dcol, b1.reshape(1, -1), W2)

    acc2 = _edge_layer(g2, e1p, e1n, cnt, layer2=True)
    a2p = a2.at[0].add(dinv[0] * dinv[0])
    logp = _tc_fin(acc2, g2, dcol, b2.reshape(1, -1), a2p.reshape(NPAD, 1),
                   W3, b3.reshape(1, -1))
    return logp[0]
